# Initial kernel scaffold; baseline (speedup 1.0000x reference)
import jax
import jax.numpy as jnp
from jax.experimental import pallas as pl


def kernel(x, edge_index, edge_weight, W1, b1, W2, b2, W3, b3):
    raise NotImplementedError("write your pallas kernel here")


# speedup vs baseline: 1.3820x; 1.3820x over previous
import functools

import jax
import jax.numpy as jnp
from jax import lax
from jax.experimental import pallas as pl
from jax.experimental.pallas import tpu as pltpu
from jax.experimental.pallas import tpu_sc as plsc

N_NODES = 10000
N_EDGES = 160000
D = 256

NC = 2
NS = 16
NW = NC * NS
RANGE = 320
N_PAD = NW * RANGE

FLUSH = 2048
PBUF = FLUSH + 80
ROW_W = 159744 + PBUF
ECHUNK = 2000
N_ECHUNK = N_EDGES // ECHUNK
KCHUNK = 64

_mesh = plsc.VectorSubcoreMesh(
    core_axis_name="c", subcore_axis_name="s", num_cores=NC, num_subcores=NS)
_sc_params = pltpu.CompilerParams(needs_layout_passes=False)


def _worker_id():
  return lax.axis_index("s") * NC + lax.axis_index("c")


@functools.partial(
    pl.kernel,
    out_type=(
        jax.ShapeDtypeStruct((NW * ROW_W,), jnp.int32),
        jax.ShapeDtypeStruct((NW * ROW_W,), jnp.int32),
        jax.ShapeDtypeStruct((NW * ROW_W,), jnp.float32),
        jax.ShapeDtypeStruct((NW * 16,), jnp.int32),
    ),
    mesh=_mesh,
    scratch_types=[
        pltpu.VMEM((ECHUNK,), jnp.int32),
        pltpu.VMEM((ECHUNK,), jnp.int32),
        pltpu.VMEM((ECHUNK,), jnp.float32),
        pltpu.VMEM((PBUF,), jnp.int32),
        pltpu.VMEM((PBUF,), jnp.int32),
        pltpu.VMEM((PBUF,), jnp.float32),
        pltpu.VMEM((16,), jnp.int32),
    ],
    compiler_params=_sc_params,
)
def _partition(dst_hbm, src_hbm, w_hbm, bsrc, bldst, bw, cnts,
               dbuf, sbuf, wbuf, pb_s, pb_l, pb_w, cvec):
  t = _worker_id()
  lo = t * RANGE

  def chunk_body(j, carry):
    eoff = pl.multiple_of(j * ECHUNK, 8)
    pltpu.sync_copy(dst_hbm.at[pl.ds(eoff, ECHUNK)], dbuf)
    pltpu.sync_copy(src_hbm.at[pl.ds(eoff, ECHUNK)], sbuf)
    pltpu.sync_copy(w_hbm.at[pl.ds(eoff, ECHUNK)], wbuf)

    def step(i, c2):
      cnt, total = c2
      d = dbuf[pl.ds(i * 16, 16)]
      sv = sbuf[pl.ds(i * 16, 16)]
      wv = wbuf[pl.ds(i * 16, 16)]
      m = (d >= lo) & (d < lo + RANGE)
      mi = m.astype(jnp.int32)
      pos = cnt + plsc.cumsum(mi) - 1
      plsc.store_scatter(pb_l, [pos], d - lo, mask=m)
      plsc.store_scatter(pb_s, [pos], sv, mask=m)
      plsc.store_scatter(pb_w, [pos], wv, mask=m)
      cnt = cnt + jnp.sum(mi)

      def do_flush(args):
        cnt, total = args
        off = pl.multiple_of(t * ROW_W + total, 8)
        pltpu.sync_copy(pb_s.at[pl.ds(0, FLUSH)], bsrc.at[pl.ds(off, FLUSH)])
        pltpu.sync_copy(pb_l.at[pl.ds(0, FLUSH)], bldst.at[pl.ds(off, FLUSH)])
        pltpu.sync_copy(pb_w.at[pl.ds(0, FLUSH)], bw.at[pl.ds(off, FLUSH)])
        rs = pb_s[pl.ds(FLUSH, 16)]
        rl = pb_l[pl.ds(FLUSH, 16)]
        rw = pb_w[pl.ds(FLUSH, 16)]
        pb_s[pl.ds(0, 16)] = rs
        pb_l[pl.ds(0, 16)] = rl
        pb_w[pl.ds(0, 16)] = rw
        return (cnt - FLUSH, total + FLUSH)

      return lax.cond(cnt >= FLUSH, do_flush, lambda a: a, (cnt, total))

    return lax.fori_loop(0, ECHUNK // 16, step, carry)

  cnt, total = lax.fori_loop(
      0, N_ECHUNK, chunk_body, (jnp.int32(0), jnp.int32(0)))

  iota = lax.iota(jnp.int32, 16)
  zi = jnp.zeros((16,), jnp.int32)
  zf = jnp.zeros((16,), jnp.float32)
  for k in range(5):
    posz = cnt + k * 16 + iota
    plsc.store_scatter(pb_l, [posz], zi)
    plsc.store_scatter(pb_s, [posz], zi)
    plsc.store_scatter(pb_w, [posz], zf)
  off = pl.multiple_of(t * ROW_W + total, 8)
  pltpu.sync_copy(pb_s.at[pl.ds(0, PBUF)], bsrc.at[pl.ds(off, PBUF)])
  pltpu.sync_copy(pb_l.at[pl.ds(0, PBUF)], bldst.at[pl.ds(off, PBUF)])
  pltpu.sync_copy(pb_w.at[pl.ds(0, PBUF)], bw.at[pl.ds(off, PBUF)])

  cvec[...] = jnp.full((16,), total + cnt, jnp.int32)
  pltpu.sync_copy(cvec, cnts.at[pl.ds(pl.multiple_of(t * 16, 8), 16)])


def _make_seg_reduce(is_max):
  @functools.partial(
      pl.kernel,
      out_type=jax.ShapeDtypeStruct((N_PAD, D), jnp.float32),
      mesh=_mesh,
      scratch_types=[
          pltpu.VMEM((RANGE, D), jnp.float32),
          pltpu.VMEM((KCHUNK, D), jnp.float32),
          pltpu.VMEM((KCHUNK,), jnp.int32),
          pltpu.VMEM((KCHUNK,), jnp.int32),
          pltpu.VMEM((KCHUNK,), jnp.float32),
          pltpu.VMEM((16,), jnp.int32),
          pltpu.SemaphoreType.DMA,
      ],
      compiler_params=_sc_params,
  )
  def seg_reduce(h_hbm, bsrc, bldst, bw, cnts, out_hbm,
                 acc, rows, ib_s, ib_l, ib_w, cbuf, sem):
    t = _worker_id()
    pltpu.sync_copy(cnts.at[pl.ds(pl.multiple_of(t * 16, 8), 16)], cbuf)
    n = jnp.max(cbuf[...])
    nchunks = (n + (KCHUNK - 1)) >> 6

    zf = jnp.zeros((16,), jnp.float32)

    def zrow(i, _):
      for dch in range(D // 16):
        acc[i, pl.ds(dch * 16, 16)] = zf
      return 0

    lax.fori_loop(0, RANGE, zrow, 0)

    iota = lax.iota(jnp.int32, 16)

    def chunk(j, _):
      base = pl.multiple_of(t * ROW_W + j * KCHUNK, 8)
      pltpu.sync_copy(bsrc.at[pl.ds(base, KCHUNK)], ib_s)
      pltpu.sync_copy(bldst.at[pl.ds(base, KCHUNK)], ib_l)
      pltpu.sync_copy(bw.at[pl.ds(base, KCHUNK)], ib_w)
      pltpu.async_copy(h_hbm.at[ib_s], rows, sem).wait()

      def edge(e, _):
        esplat = jnp.full((16,), e, jnp.int32)
        ew = plsc.load_gather(ib_w, [esplat])
        el = plsc.load_gather(ib_l, [esplat])
        for dch in range(D // 16):
          col = dch * 16 + iota
          msg = rows[e, pl.ds(dch * 16, 16)] * ew
          if is_max:
            cur = plsc.load_gather(acc, [el, col])
            plsc.store_scatter(acc, [el, col], jnp.maximum(cur, msg))
          else:
            plsc.addupdate_scatter(acc, [el, col], msg)
        return 0

      lax.fori_loop(0, KCHUNK, edge, 0)
      return 0

    lax.fori_loop(0, nchunks, chunk, 0)
    pltpu.sync_copy(acc, out_hbm.at[pl.ds(pl.multiple_of(t * RANGE, 8), RANGE)])

  return seg_reduce


_seg_sum = _make_seg_reduce(is_max=False)
_seg_max_agg = _make_seg_reduce(is_max=True)


MM_BLOCK = 1000


def _mlp_body(x_ref, agg_ref, w_ref, b_ref, o_ref, *, relu):
  s = x_ref[...] + agg_ref[...]
  o = jnp.dot(s, w_ref[...], preferred_element_type=jnp.float32) + b_ref[...]
  if relu:
    o = jnp.maximum(o, 0.0)
  o_ref[...] = o


def _mlp(x, agg, w, b, relu):
  return pl.pallas_call(
      functools.partial(_mlp_body, relu=relu),
      grid=(N_NODES // MM_BLOCK,),
      in_specs=[
          pl.BlockSpec((MM_BLOCK, D), lambda i: (i, 0)),
          pl.BlockSpec((MM_BLOCK, D), lambda i: (i, 0)),
          pl.BlockSpec((D, D), lambda i: (0, 0)),
          pl.BlockSpec((1, D), lambda i: (0, 0)),
      ],
      out_specs=pl.BlockSpec((MM_BLOCK, D), lambda i: (i, 0)),
      out_shape=jax.ShapeDtypeStruct((N_NODES, D), jnp.float32),
  )(x, agg, w, b.reshape(1, D))


def kernel(x, edge_index, edge_weight, W1, b1, W2, b2, W3, b3):
  src = edge_index[0]
  dst = edge_index[1]
  bsrc, bldst, bw, cnts = _partition(dst, src, edge_weight)

  agg1 = _seg_sum(x, bsrc, bldst, bw, cnts)[:N_NODES]
  h = _mlp(x, agg1, W1, b1, relu=True)
  agg2 = _seg_max_agg(h, bsrc, bldst, bw, cnts)[:N_NODES]
  h = _mlp(h, agg2, W2, b2, relu=True)
  agg3 = _seg_max_agg(h, bsrc, bldst, bw, cnts)[:N_NODES]
  return _mlp(h, agg3, W3, b3, relu=False)


# speedup vs baseline: 1.8403x; 1.3316x over previous
import functools

import jax
import jax.numpy as jnp
from jax import lax
from jax.experimental import pallas as pl
from jax.experimental.pallas import tpu as pltpu
from jax.experimental.pallas import tpu_sc as plsc

N_NODES = 10000
N_EDGES = 160000
D = 256

NC = 2
NS = 16
NW = NC * NS
RANGE = 320
N_PAD = NW * RANGE

ESLOT = 2048
ISLOT = 128
FLUSH = 2048
PBUF = FLUSH + 80
ROW_W = 159744 + PBUF
ECHUNK = 2000
N_ECHUNK = N_EDGES // ECHUNK
KCHUNK = 64

_mesh = plsc.VectorSubcoreMesh(
    core_axis_name="c", subcore_axis_name="s", num_cores=NC, num_subcores=NS)
_sc_params = pltpu.CompilerParams(needs_layout_passes=False)


def _worker_id():
  return lax.axis_index("s") * NC + lax.axis_index("c")


@functools.partial(
    pl.kernel,
    out_type=(
        jax.ShapeDtypeStruct((NW * ROW_W,), jnp.int32),
        jax.ShapeDtypeStruct((NW * ROW_W,), jnp.int32),
        jax.ShapeDtypeStruct((NW * ROW_W,), jnp.float32),
        jax.ShapeDtypeStruct((NW * 16,), jnp.int32),
    ),
    mesh=_mesh,
    scratch_types=[
        pltpu.VMEM((2 * ESLOT,), jnp.int32),
        pltpu.VMEM((2 * ESLOT,), jnp.int32),
        pltpu.VMEM((2 * ESLOT,), jnp.float32),
        pltpu.VMEM((PBUF,), jnp.int32),
        pltpu.VMEM((PBUF,), jnp.int32),
        pltpu.VMEM((PBUF,), jnp.float32),
        pltpu.VMEM((16,), jnp.int32),
        pltpu.SemaphoreType.DMA((2,)),
    ],
    compiler_params=_sc_params,
)
def _partition(dst_hbm, src_hbm, w_hbm, bsrc, bldst, bw, cnts,
               dbuf, sbuf, wbuf, pb_s, pb_l, pb_w, cvec, semc):
  t = _worker_id()
  lo = t * RANGE

  def in_copies(j, slot):
    eoff = pl.multiple_of(j * ECHUNK, 8)
    sbase = pl.multiple_of(slot * ESLOT, 8)
    return (
        pltpu.make_async_copy(dst_hbm.at[pl.ds(eoff, ECHUNK)],
                              dbuf.at[pl.ds(sbase, ECHUNK)], semc.at[slot]),
        pltpu.make_async_copy(src_hbm.at[pl.ds(eoff, ECHUNK)],
                              sbuf.at[pl.ds(sbase, ECHUNK)], semc.at[slot]),
        pltpu.make_async_copy(w_hbm.at[pl.ds(eoff, ECHUNK)],
                              wbuf.at[pl.ds(sbase, ECHUNK)], semc.at[slot]),
    )

  for c in in_copies(0, 0):
    c.start()

  def chunk_body(j, carry):
    slot = j & 1
    nxt = 1 - slot

    @pl.when(j + 1 < N_ECHUNK)
    def _():
      for c in in_copies(j + 1, nxt):
        c.start()

    for c in in_copies(j, slot):
      c.wait()

    sbase = slot * ESLOT

    def step(i, c2):
      cnt, total = c2
      d = dbuf[pl.ds(sbase + i * 16, 16)]
      sv = sbuf[pl.ds(sbase + i * 16, 16)]
      wv = wbuf[pl.ds(sbase + i * 16, 16)]
      m = (d >= lo) & (d < lo + RANGE)
      mi = m.astype(jnp.int32)
      pos = cnt + plsc.cumsum(mi) - 1
      plsc.store_scatter(pb_l, [pos], d - lo, mask=m)
      plsc.store_scatter(pb_s, [pos], sv, mask=m)
      plsc.store_scatter(pb_w, [pos], wv, mask=m)
      cnt = cnt + jnp.sum(mi)

      def do_flush(args):
        cnt, total = args
        off = pl.multiple_of(t * ROW_W + total, 8)
        pltpu.sync_copy(pb_s.at[pl.ds(0, FLUSH)], bsrc.at[pl.ds(off, FLUSH)])
        pltpu.sync_copy(pb_l.at[pl.ds(0, FLUSH)], bldst.at[pl.ds(off, FLUSH)])
        pltpu.sync_copy(pb_w.at[pl.ds(0, FLUSH)], bw.at[pl.ds(off, FLUSH)])
        rs = pb_s[pl.ds(FLUSH, 16)]
        rl = pb_l[pl.ds(FLUSH, 16)]
        rw = pb_w[pl.ds(FLUSH, 16)]
        pb_s[pl.ds(0, 16)] = rs
        pb_l[pl.ds(0, 16)] = rl
        pb_w[pl.ds(0, 16)] = rw
        return (cnt - FLUSH, total + FLUSH)

      return lax.cond(cnt >= FLUSH, do_flush, lambda a: a, (cnt, total))

    return lax.fori_loop(0, ECHUNK // 16, step, carry)

  cnt, total = lax.fori_loop(
      0, N_ECHUNK, chunk_body, (jnp.int32(0), jnp.int32(0)))

  iota = lax.iota(jnp.int32, 16)
  zi = jnp.zeros((16,), jnp.int32)
  zf = jnp.zeros((16,), jnp.float32)
  for k in range(5):
    posz = cnt + k * 16 + iota
    plsc.store_scatter(pb_l, [posz], zi)
    plsc.store_scatter(pb_s, [posz], zi)
    plsc.store_scatter(pb_w, [posz], zf)
  off = pl.multiple_of(t * ROW_W + total, 8)
  pltpu.sync_copy(pb_s.at[pl.ds(0, PBUF)], bsrc.at[pl.ds(off, PBUF)])
  pltpu.sync_copy(pb_l.at[pl.ds(0, PBUF)], bldst.at[pl.ds(off, PBUF)])
  pltpu.sync_copy(pb_w.at[pl.ds(0, PBUF)], bw.at[pl.ds(off, PBUF)])

  cvec[...] = jnp.full((16,), total + cnt, jnp.int32)
  pltpu.sync_copy(cvec, cnts.at[pl.ds(pl.multiple_of(t * 16, 8), 16)])


EUNROLL = 4


def _make_seg_reduce(is_max):
  @functools.partial(
      pl.kernel,
      out_type=jax.ShapeDtypeStruct((N_PAD, D), jnp.float32),
      mesh=_mesh,
      scratch_types=[
          pltpu.VMEM((RANGE, D), jnp.float32),
          pltpu.VMEM((2, KCHUNK, D), jnp.float32),
          pltpu.VMEM((2 * ISLOT,), jnp.int32),
          pltpu.VMEM((2 * ISLOT,), jnp.int32),
          pltpu.VMEM((2 * ISLOT,), jnp.float32),
          pltpu.VMEM((16,), jnp.int32),
          pltpu.SemaphoreType.DMA((2,)),
          pltpu.SemaphoreType.DMA((2,)),
      ],
      compiler_params=_sc_params,
  )
  def seg_reduce(h_hbm, bsrc, bldst, bw, cnts, out_hbm,
                 acc, rows, ib_s, ib_l, ib_w, cbuf, semi, semr):
    t = _worker_id()
    pltpu.sync_copy(cnts.at[pl.ds(pl.multiple_of(t * 16, 8), 16)], cbuf)
    n = jnp.max(cbuf[...])
    nchunks = (n + (KCHUNK - 1)) >> 6

    def idx_copies(j, slot):
      base = pl.multiple_of(t * ROW_W + j * KCHUNK, 8)
      sb = pl.multiple_of(slot * ISLOT, 8)
      return (
          pltpu.make_async_copy(bsrc.at[pl.ds(base, KCHUNK)],
                                ib_s.at[pl.ds(sb, KCHUNK)], semi.at[slot]),
          pltpu.make_async_copy(bldst.at[pl.ds(base, KCHUNK)],
                                ib_l.at[pl.ds(sb, KCHUNK)], semi.at[slot]),
          pltpu.make_async_copy(bw.at[pl.ds(base, KCHUNK)],
                                ib_w.at[pl.ds(sb, KCHUNK)], semi.at[slot]),
      )

    def row_copy(slot):
      sb = pl.multiple_of(slot * ISLOT, 8)
      return pltpu.make_async_copy(h_hbm.at[ib_s.at[pl.ds(sb, KCHUNK)]],
                                   rows.at[slot], semr.at[slot])

    zf = jnp.zeros((16,), jnp.float32)

    @pl.when(nchunks > 0)
    def _():
      for c in idx_copies(0, 0):
        c.start()

    def zrow(i, _):
      for dch in range(D // 16):
        acc[i, pl.ds(dch * 16, 16)] = zf
      return 0

    lax.fori_loop(0, RANGE, zrow, 0)

    @pl.when(nchunks > 0)
    def _():
      for c in idx_copies(0, 0):
        c.wait()
      row_copy(0).start()

    @pl.when(nchunks > 1)
    def _():
      for c in idx_copies(1, 1):
        c.start()

    iota = lax.iota(jnp.int32, 16)

    def chunk(j, _):
      slot = j & 1
      nxt = 1 - slot

      @pl.when(j + 1 < nchunks)
      def _():
        for c in idx_copies(j + 1, nxt):
          c.wait()
        row_copy(nxt).start()

      row_copy(slot).wait()

      sb = slot * ISLOT

      def edge(q, _):
        for u in range(EUNROLL):
          e = q * EUNROLL + u
          esplat = jnp.full((16,), sb + e, jnp.int32)
          ew = plsc.load_gather(ib_w, [esplat])
          el = plsc.load_gather(ib_l, [esplat])
          for dch in range(D // 16):
            col = dch * 16 + iota
            msg = rows[slot, e, pl.ds(dch * 16, 16)] * ew
            if is_max:
              cur = plsc.load_gather(acc, [el, col])
              plsc.store_scatter(acc, [el, col], jnp.maximum(cur, msg))
            else:
              plsc.addupdate_scatter(acc, [el, col], msg)
        return 0

      lax.fori_loop(0, KCHUNK // EUNROLL, edge, 0)

      @pl.when(j + 2 < nchunks)
      def _():
        for c in idx_copies(j + 2, slot):
          c.start()

      return 0

    lax.fori_loop(0, nchunks, chunk, 0)
    pltpu.sync_copy(acc, out_hbm.at[pl.ds(pl.multiple_of(t * RANGE, 8), RANGE)])

  return seg_reduce


_seg_sum = _make_seg_reduce(is_max=False)
_seg_max_agg = _make_seg_reduce(is_max=True)


MM_BLOCK = 1000


def _mlp_body(x_ref, agg_ref, w_ref, b_ref, o_ref, *, relu):
  s = x_ref[...] + agg_ref[...]
  o = jnp.dot(s, w_ref[...], preferred_element_type=jnp.float32) + b_ref[...]
  if relu:
    o = jnp.maximum(o, 0.0)
  o_ref[...] = o


def _mlp(x, agg, w, b, relu):
  return pl.pallas_call(
      functools.partial(_mlp_body, relu=relu),
      grid=(N_NODES // MM_BLOCK,),
      in_specs=[
          pl.BlockSpec((MM_BLOCK, D), lambda i: (i, 0)),
          pl.BlockSpec((MM_BLOCK, D), lambda i: (i, 0)),
          pl.BlockSpec((D, D), lambda i: (0, 0)),
          pl.BlockSpec((1, D), lambda i: (0, 0)),
      ],
      out_specs=pl.BlockSpec((MM_BLOCK, D), lambda i: (i, 0)),
      out_shape=jax.ShapeDtypeStruct((N_NODES, D), jnp.float32),
  )(x, agg, w, b.reshape(1, D))


def kernel(x, edge_index, edge_weight, W1, b1, W2, b2, W3, b3):
  src = edge_index[0]
  dst = edge_index[1]
  bsrc, bldst, bw, cnts = _partition(dst, src, edge_weight)

  agg1 = _seg_sum(x, bsrc, bldst, bw, cnts)[:N_NODES]
  h = _mlp(x, agg1, W1, b1, relu=True)
  agg2 = _seg_max_agg(h, bsrc, bldst, bw, cnts)[:N_NODES]
  h = _mlp(h, agg2, W2, b2, relu=True)
  agg3 = _seg_max_agg(h, bsrc, bldst, bw, cnts)[:N_NODES]
  return _mlp(h, agg3, W3, b3, relu=False)


# speedup vs baseline: 1.9226x; 1.0448x over previous
import functools

import jax
import jax.numpy as jnp
from jax import lax
from jax.experimental import pallas as pl
from jax.experimental.pallas import tpu as pltpu
from jax.experimental.pallas import tpu_sc as plsc

N_NODES = 10000
N_EDGES = 160000
D = 256

NC = 2
NS = 16
NW = NC * NS
RANGE = 320
N_PAD = NW * RANGE

ESLOT = 2048
ISLOT = 128
FLUSH = 2048
PUNROLL = 5
PBUF = 4128
FFLUSH = FLUSH + 80
ROW_W = 159744 + FFLUSH
ECHUNK = 2000
N_ECHUNK = N_EDGES // ECHUNK
KCHUNK = 64

_mesh = plsc.VectorSubcoreMesh(
    core_axis_name="c", subcore_axis_name="s", num_cores=NC, num_subcores=NS)
_sc_params = pltpu.CompilerParams(needs_layout_passes=False)


def _worker_id():
  return lax.axis_index("s") * NC + lax.axis_index("c")


@functools.partial(
    pl.kernel,
    out_type=(
        jax.ShapeDtypeStruct((NW * ROW_W,), jnp.int32),
        jax.ShapeDtypeStruct((NW * ROW_W,), jnp.int32),
        jax.ShapeDtypeStruct((NW * ROW_W,), jnp.float32),
        jax.ShapeDtypeStruct((NW * 16,), jnp.int32),
    ),
    mesh=_mesh,
    scratch_types=[
        pltpu.VMEM((2 * ESLOT,), jnp.int32),
        pltpu.VMEM((2 * ESLOT,), jnp.int32),
        pltpu.VMEM((2 * ESLOT,), jnp.float32),
        pltpu.VMEM((PBUF,), jnp.int32),
        pltpu.VMEM((PBUF,), jnp.int32),
        pltpu.VMEM((PBUF,), jnp.float32),
        pltpu.VMEM((16,), jnp.int32),
        pltpu.SemaphoreType.DMA((2,)),
    ],
    compiler_params=_sc_params,
)
def _partition(dst_hbm, src_hbm, w_hbm, bsrc, bldst, bw, cnts,
               dbuf, sbuf, wbuf, pb_s, pb_l, pb_w, cvec, semc):
  t = _worker_id()
  lo = t * RANGE

  def in_copies(j, slot):
    eoff = pl.multiple_of(j * ECHUNK, 8)
    sbase = pl.multiple_of(slot * ESLOT, 8)
    return (
        pltpu.make_async_copy(dst_hbm.at[pl.ds(eoff, ECHUNK)],
                              dbuf.at[pl.ds(sbase, ECHUNK)], semc.at[slot]),
        pltpu.make_async_copy(src_hbm.at[pl.ds(eoff, ECHUNK)],
                              sbuf.at[pl.ds(sbase, ECHUNK)], semc.at[slot]),
        pltpu.make_async_copy(w_hbm.at[pl.ds(eoff, ECHUNK)],
                              wbuf.at[pl.ds(sbase, ECHUNK)], semc.at[slot]),
    )

  for c in in_copies(0, 0):
    c.start()

  def chunk_body(j, carry):
    slot = j & 1
    nxt = 1 - slot

    @pl.when(j + 1 < N_ECHUNK)
    def _():
      for c in in_copies(j + 1, nxt):
        c.start()

    for c in in_copies(j, slot):
      c.wait()

    sbase = slot * ESLOT
    last16 = jnp.full((16,), 15, jnp.int32)
    cnt_vec, total = carry

    def step(q, cnt_vec):
      for u in range(PUNROLL):
        i = q * PUNROLL + u
        d = dbuf[pl.ds(sbase + i * 16, 16)]
        sv = sbuf[pl.ds(sbase + i * 16, 16)]
        wv = wbuf[pl.ds(sbase + i * 16, 16)]
        m = (d >= lo) & (d < lo + RANGE)
        mi = m.astype(jnp.int32)
        pos = cnt_vec + plsc.cumsum(mi) - 1
        plsc.store_scatter(pb_l, [pos], d - lo, mask=m)
        plsc.store_scatter(pb_s, [pos], sv, mask=m)
        plsc.store_scatter(pb_w, [pos], wv, mask=m)
        cnt_vec = jnp.full((16,), pos[15] + 1, jnp.int32)
      return cnt_vec

    cnt_vec = lax.fori_loop(0, ECHUNK // 16 // PUNROLL, step, cnt_vec)

    cnt_s = jnp.max(cnt_vec)

    def do_flush(args):
      cnt_s, cnt_vec, total = args
      off = pl.multiple_of(t * ROW_W + total, 8)
      pltpu.sync_copy(pb_s.at[pl.ds(0, FLUSH)], bsrc.at[pl.ds(off, FLUSH)])
      pltpu.sync_copy(pb_l.at[pl.ds(0, FLUSH)], bldst.at[pl.ds(off, FLUSH)])
      pltpu.sync_copy(pb_w.at[pl.ds(0, FLUSH)], bw.at[pl.ds(off, FLUSH)])
      rem = cnt_s - FLUSH

      def mv(i, _):
        rs = pb_s[pl.ds(FLUSH + i * 16, 16)]
        rl = pb_l[pl.ds(FLUSH + i * 16, 16)]
        rw = pb_w[pl.ds(FLUSH + i * 16, 16)]
        pb_s[pl.ds(i * 16, 16)] = rs
        pb_l[pl.ds(i * 16, 16)] = rl
        pb_w[pl.ds(i * 16, 16)] = rw
        return 0

      lax.fori_loop(0, (rem + 15) >> 4, mv, 0)
      return (cnt_vec - FLUSH, total + FLUSH)

    return lax.cond(cnt_s >= FLUSH, do_flush,
                    lambda a: (a[1], a[2]), (cnt_s, cnt_vec, total))

  cnt_vec, total = lax.fori_loop(
      0, N_ECHUNK, chunk_body, (jnp.zeros((16,), jnp.int32), jnp.int32(0)))
  cnt = jnp.max(cnt_vec)

  iota = lax.iota(jnp.int32, 16)
  zi = jnp.zeros((16,), jnp.int32)
  zf = jnp.zeros((16,), jnp.float32)
  for k in range(5):
    posz = cnt + k * 16 + iota
    plsc.store_scatter(pb_l, [posz], zi)
    plsc.store_scatter(pb_s, [posz], zi)
    plsc.store_scatter(pb_w, [posz], zf)
  off = pl.multiple_of(t * ROW_W + total, 8)
  pltpu.sync_copy(pb_s.at[pl.ds(0, FFLUSH)], bsrc.at[pl.ds(off, FFLUSH)])
  pltpu.sync_copy(pb_l.at[pl.ds(0, FFLUSH)], bldst.at[pl.ds(off, FFLUSH)])
  pltpu.sync_copy(pb_w.at[pl.ds(0, FFLUSH)], bw.at[pl.ds(off, FFLUSH)])

  cvec[...] = jnp.full((16,), total + cnt, jnp.int32)
  pltpu.sync_copy(cvec, cnts.at[pl.ds(pl.multiple_of(t * 16, 8), 16)])


EUNROLL = 4


def _make_seg_reduce(is_max):
  @functools.partial(
      pl.kernel,
      out_type=jax.ShapeDtypeStruct((N_PAD, D), jnp.float32),
      mesh=_mesh,
      scratch_types=[
          pltpu.VMEM((RANGE, D), jnp.float32),
          pltpu.VMEM((2, KCHUNK, D), jnp.float32),
          pltpu.VMEM((2 * ISLOT,), jnp.int32),
          pltpu.VMEM((2 * ISLOT,), jnp.int32),
          pltpu.VMEM((2 * ISLOT,), jnp.float32),
          pltpu.VMEM((16,), jnp.int32),
          pltpu.SemaphoreType.DMA((2,)),
          pltpu.SemaphoreType.DMA((2,)),
      ],
      compiler_params=_sc_params,
  )
  def seg_reduce(h_hbm, bsrc, bldst, bw, cnts, out_hbm,
                 acc, rows, ib_s, ib_l, ib_w, cbuf, semi, semr):
    t = _worker_id()
    pltpu.sync_copy(cnts.at[pl.ds(pl.multiple_of(t * 16, 8), 16)], cbuf)
    n = jnp.max(cbuf[...])
    nchunks = (n + (KCHUNK - 1)) >> 6

    def idx_copies(j, slot):
      base = pl.multiple_of(t * ROW_W + j * KCHUNK, 8)
      sb = pl.multiple_of(slot * ISLOT, 8)
      return (
          pltpu.make_async_copy(bsrc.at[pl.ds(base, KCHUNK)],
                                ib_s.at[pl.ds(sb, KCHUNK)], semi.at[slot]),
          pltpu.make_async_copy(bldst.at[pl.ds(base, KCHUNK)],
                                ib_l.at[pl.ds(sb, KCHUNK)], semi.at[slot]),
          pltpu.make_async_copy(bw.at[pl.ds(base, KCHUNK)],
                                ib_w.at[pl.ds(sb, KCHUNK)], semi.at[slot]),
      )

    def row_copy(slot):
      sb = pl.multiple_of(slot * ISLOT, 8)
      return pltpu.make_async_copy(h_hbm.at[ib_s.at[pl.ds(sb, KCHUNK)]],
                                   rows.at[slot], semr.at[slot])

    zf = jnp.zeros((16,), jnp.float32)

    @pl.when(nchunks > 0)
    def _():
      for c in idx_copies(0, 0):
        c.start()

    def zrow(i, _):
      for dch in range(D // 16):
        acc[i, pl.ds(dch * 16, 16)] = zf
      return 0

    lax.fori_loop(0, RANGE, zrow, 0)

    @pl.when(nchunks > 0)
    def _():
      for c in idx_copies(0, 0):
        c.wait()
      row_copy(0).start()

    @pl.when(nchunks > 1)
    def _():
      for c in idx_copies(1, 1):
        c.start()

    iota = lax.iota(jnp.int32, 16)

    def chunk(j, _):
      slot = j & 1
      nxt = 1 - slot

      @pl.when(j + 1 < nchunks)
      def _():
        for c in idx_copies(j + 1, nxt):
          c.wait()
        row_copy(nxt).start()

      row_copy(slot).wait()

      sb = slot * ISLOT

      def edge_group(g, _):
        lvec = ib_l[pl.ds(sb + g * 16, 16)]
        wvec = ib_w[pl.ds(sb + g * 16, 16)]
        for u in range(16):
          row = lvec[u]
          wsc = wvec[u]
          e = g * 16 + u
          for dch in range(D // 16):
            c0 = dch * 16
            msg = rows[slot, e, pl.ds(c0, 16)] * wsc
            cur = acc[row, pl.ds(c0, 16)]
            if is_max:
              acc[row, pl.ds(c0, 16)] = jnp.maximum(cur, msg)
            else:
              acc[row, pl.ds(c0, 16)] = cur + msg
        return 0

      lax.fori_loop(0, KCHUNK // 16, edge_group, 0)

      @pl.when(j + 2 < nchunks)
      def _():
        for c in idx_copies(j + 2, slot):
          c.start()

      return 0

    lax.fori_loop(0, nchunks, chunk, 0)
    pltpu.sync_copy(acc, out_hbm.at[pl.ds(pl.multiple_of(t * RANGE, 8), RANGE)])

  return seg_reduce


_seg_sum = _make_seg_reduce(is_max=False)
_seg_max_agg = _make_seg_reduce(is_max=True)


MM_BLOCK = 1000


def _mlp_body(x_ref, agg_ref, w_ref, b_ref, o_ref, *, relu):
  s = x_ref[...] + agg_ref[...]
  o = jnp.dot(s, w_ref[...], preferred_element_type=jnp.float32) + b_ref[...]
  if relu:
    o = jnp.maximum(o, 0.0)
  o_ref[...] = o


def _mlp(x, agg, w, b, relu):
  return pl.pallas_call(
      functools.partial(_mlp_body, relu=relu),
      grid=(N_NODES // MM_BLOCK,),
      in_specs=[
          pl.BlockSpec((MM_BLOCK, D), lambda i: (i, 0)),
          pl.BlockSpec((MM_BLOCK, D), lambda i: (i, 0)),
          pl.BlockSpec((D, D), lambda i: (0, 0)),
          pl.BlockSpec((1, D), lambda i: (0, 0)),
      ],
      out_specs=pl.BlockSpec((MM_BLOCK, D), lambda i: (i, 0)),
      out_shape=jax.ShapeDtypeStruct((N_NODES, D), jnp.float32),
  )(x, agg, w, b.reshape(1, D))


def kernel(x, edge_index, edge_weight, W1, b1, W2, b2, W3, b3):
  src = edge_index[0]
  dst = edge_index[1]
  bsrc, bldst, bw, cnts = _partition(dst, src, edge_weight)

  agg1 = _seg_sum(x, bsrc, bldst, bw, cnts)[:N_NODES]
  h = _mlp(x, agg1, W1, b1, relu=True)
  agg2 = _seg_max_agg(h, bsrc, bldst, bw, cnts)[:N_NODES]
  h = _mlp(h, agg2, W2, b2, relu=True)
  agg3 = _seg_max_agg(h, bsrc, bldst, bw, cnts)[:N_NODES]
  return _mlp(h, agg3, W3, b3, relu=False)


# speedup vs baseline: 4.2478x; 2.2093x over previous
import functools

import jax
import jax.numpy as jnp
from jax import lax
from jax.experimental import pallas as pl
from jax.experimental.pallas import tpu as pltpu
from jax.experimental.pallas import tpu_sc as plsc

N_NODES = 10000
N_EDGES = 160000
D = 256

NC = 2
NS = 16
NW = NC * NS
RANGE = 320
N_PAD = NW * RANGE

ESLOT = 2048
ISLOT = 128
FLUSH = 2048
PUNROLL = 5
PBUF = 4128
FFLUSH = FLUSH + 80
ROW_W = 159744 + FFLUSH
ECHUNK = 2000
N_ECHUNK = N_EDGES // ECHUNK
KCHUNK = 64

_mesh = plsc.VectorSubcoreMesh(
    core_axis_name="c", subcore_axis_name="s", num_cores=NC, num_subcores=NS)
_sc_params = pltpu.CompilerParams(needs_layout_passes=False)


def _worker_id():
  return lax.axis_index("s") * NC + lax.axis_index("c")


@functools.partial(
    pl.kernel,
    out_type=(
        jax.ShapeDtypeStruct((NW * ROW_W,), jnp.int32),
        jax.ShapeDtypeStruct((NW * ROW_W,), jnp.int32),
        jax.ShapeDtypeStruct((NW * ROW_W,), jnp.float32),
        jax.ShapeDtypeStruct((NW * 16,), jnp.int32),
    ),
    mesh=_mesh,
    scratch_types=[
        pltpu.VMEM((2 * ESLOT,), jnp.int32),
        pltpu.VMEM((2 * ESLOT,), jnp.int32),
        pltpu.VMEM((2 * ESLOT,), jnp.float32),
        pltpu.VMEM((PBUF,), jnp.int32),
        pltpu.VMEM((PBUF,), jnp.int32),
        pltpu.VMEM((PBUF,), jnp.float32),
        pltpu.VMEM((16,), jnp.int32),
        pltpu.SemaphoreType.DMA((2,)),
    ],
    compiler_params=_sc_params,
)
def _partition(dst_hbm, src_hbm, w_hbm, bsrc, bldst, bw, cnts,
               dbuf, sbuf, wbuf, pb_s, pb_l, pb_w, cvec, semc):
  t = _worker_id()
  lo = t * RANGE

  def in_copies(j, slot):
    eoff = pl.multiple_of(j * ECHUNK, 8)
    sbase = pl.multiple_of(slot * ESLOT, 8)
    return (
        pltpu.make_async_copy(dst_hbm.at[pl.ds(eoff, ECHUNK)],
                              dbuf.at[pl.ds(sbase, ECHUNK)], semc.at[slot]),
        pltpu.make_async_copy(src_hbm.at[pl.ds(eoff, ECHUNK)],
                              sbuf.at[pl.ds(sbase, ECHUNK)], semc.at[slot]),
        pltpu.make_async_copy(w_hbm.at[pl.ds(eoff, ECHUNK)],
                              wbuf.at[pl.ds(sbase, ECHUNK)], semc.at[slot]),
    )

  for c in in_copies(0, 0):
    c.start()

  def chunk_body(j, carry):
    slot = j & 1
    nxt = 1 - slot

    @pl.when(j + 1 < N_ECHUNK)
    def _():
      for c in in_copies(j + 1, nxt):
        c.start()

    for c in in_copies(j, slot):
      c.wait()

    sbase = slot * ESLOT
    last16 = jnp.full((16,), 15, jnp.int32)
    cnt_vec, total = carry

    def step(q, cnt_vec):
      for u in range(PUNROLL):
        i = q * PUNROLL + u
        d = dbuf[pl.ds(sbase + i * 16, 16)]
        sv = sbuf[pl.ds(sbase + i * 16, 16)]
        wv = wbuf[pl.ds(sbase + i * 16, 16)]
        m = (d >= lo) & (d < lo + RANGE)
        mi = m.astype(jnp.int32)
        pos = cnt_vec + plsc.cumsum(mi) - 1
        plsc.store_scatter(pb_l, [pos], d - lo, mask=m)
        plsc.store_scatter(pb_s, [pos], sv, mask=m)
        plsc.store_scatter(pb_w, [pos], wv, mask=m)
        cnt_vec = jnp.full((16,), pos[15] + 1, jnp.int32)
      return cnt_vec

    cnt_vec = lax.fori_loop(0, ECHUNK // 16 // PUNROLL, step, cnt_vec)

    cnt_s = jnp.max(cnt_vec)

    def do_flush(args):
      cnt_s, cnt_vec, total = args
      off = pl.multiple_of(t * ROW_W + total, 8)
      pltpu.sync_copy(pb_s.at[pl.ds(0, FLUSH)], bsrc.at[pl.ds(off, FLUSH)])
      pltpu.sync_copy(pb_l.at[pl.ds(0, FLUSH)], bldst.at[pl.ds(off, FLUSH)])
      pltpu.sync_copy(pb_w.at[pl.ds(0, FLUSH)], bw.at[pl.ds(off, FLUSH)])
      rem = cnt_s - FLUSH

      def mv(i, _):
        rs = pb_s[pl.ds(FLUSH + i * 16, 16)]
        rl = pb_l[pl.ds(FLUSH + i * 16, 16)]
        rw = pb_w[pl.ds(FLUSH + i * 16, 16)]
        pb_s[pl.ds(i * 16, 16)] = rs
        pb_l[pl.ds(i * 16, 16)] = rl
        pb_w[pl.ds(i * 16, 16)] = rw
        return 0

      lax.fori_loop(0, (rem + 15) >> 4, mv, 0)
      return (cnt_vec - FLUSH, total + FLUSH)

    return lax.cond(cnt_s >= FLUSH, do_flush,
                    lambda a: (a[1], a[2]), (cnt_s, cnt_vec, total))

  cnt_vec, total = lax.fori_loop(
      0, N_ECHUNK, chunk_body, (jnp.zeros((16,), jnp.int32), jnp.int32(0)))
  cnt = jnp.max(cnt_vec)

  iota = lax.iota(jnp.int32, 16)
  zi = jnp.zeros((16,), jnp.int32)
  zf = jnp.zeros((16,), jnp.float32)
  for k in range(5):
    posz = cnt + k * 16 + iota
    plsc.store_scatter(pb_l, [posz], zi)
    plsc.store_scatter(pb_s, [posz], zi)
    plsc.store_scatter(pb_w, [posz], zf)
  off = pl.multiple_of(t * ROW_W + total, 8)
  pltpu.sync_copy(pb_s.at[pl.ds(0, FFLUSH)], bsrc.at[pl.ds(off, FFLUSH)])
  pltpu.sync_copy(pb_l.at[pl.ds(0, FFLUSH)], bldst.at[pl.ds(off, FFLUSH)])
  pltpu.sync_copy(pb_w.at[pl.ds(0, FFLUSH)], bw.at[pl.ds(off, FFLUSH)])

  cvec[...] = jnp.full((16,), total + cnt, jnp.int32)
  pltpu.sync_copy(cvec, cnts.at[pl.ds(pl.multiple_of(t * 16, 8), 16)])


EUNROLL = 4


def _make_seg_reduce(is_max):
  @functools.partial(
      pl.kernel,
      out_type=jax.ShapeDtypeStruct((N_PAD, D), jnp.float32),
      mesh=_mesh,
      scratch_types=[
          pltpu.VMEM((RANGE, D), jnp.float32),
          pltpu.VMEM((2, KCHUNK, D), jnp.float32),
          pltpu.VMEM((2 * ISLOT,), jnp.int32),
          pltpu.VMEM((2 * ISLOT,), jnp.int32),
          pltpu.VMEM((2 * ISLOT,), jnp.float32),
          pltpu.VMEM((16,), jnp.int32),
          pltpu.SemaphoreType.DMA((2,)),
          pltpu.SemaphoreType.DMA((2,)),
      ],
      compiler_params=_sc_params,
  )
  def seg_reduce(h_hbm, bsrc, bldst, bw, cnts, out_hbm,
                 acc, rows, ib_s, ib_l, ib_w, cbuf, semi, semr):
    t = _worker_id()
    pltpu.sync_copy(cnts.at[pl.ds(pl.multiple_of(t * 16, 8), 16)], cbuf)
    n = jnp.max(cbuf[...])
    nchunks = (n + (KCHUNK - 1)) >> 6

    def idx_copies(j, slot):
      base = pl.multiple_of(t * ROW_W + j * KCHUNK, 8)
      sb = pl.multiple_of(slot * ISLOT, 8)
      return (
          pltpu.make_async_copy(bsrc.at[pl.ds(base, KCHUNK)],
                                ib_s.at[pl.ds(sb, KCHUNK)], semi.at[slot]),
          pltpu.make_async_copy(bldst.at[pl.ds(base, KCHUNK)],
                                ib_l.at[pl.ds(sb, KCHUNK)], semi.at[slot]),
          pltpu.make_async_copy(bw.at[pl.ds(base, KCHUNK)],
                                ib_w.at[pl.ds(sb, KCHUNK)], semi.at[slot]),
      )

    def row_copy(slot):
      sb = pl.multiple_of(slot * ISLOT, 8)
      return pltpu.make_async_copy(h_hbm.at[ib_s.at[pl.ds(sb, KCHUNK)]],
                                   rows.at[slot], semr.at[slot])

    zf = jnp.zeros((16,), jnp.float32)

    @pl.when(nchunks > 0)
    def _():
      for c in idx_copies(0, 0):
        c.start()

    def zrow(i, _):
      for dch in range(D // 16):
        acc[i, pl.ds(dch * 16, 16)] = zf
      return 0

    lax.fori_loop(0, RANGE, zrow, 0)

    @pl.when(nchunks > 0)
    def _():
      for c in idx_copies(0, 0):
        c.wait()
      row_copy(0).start()

    @pl.when(nchunks > 1)
    def _():
      for c in idx_copies(1, 1):
        c.start()

    iota = lax.iota(jnp.int32, 16)

    def chunk(j, _):
      slot = j & 1
      nxt = 1 - slot

      @pl.when(j + 1 < nchunks)
      def _():
        for c in idx_copies(j + 1, nxt):
          c.wait()
        row_copy(nxt).start()

      row_copy(slot).wait()

      sb = slot * ISLOT

      def edge_group(g, _):
        lvec = ib_l[pl.ds(sb + g * 16, 16)]
        wvec = ib_w[pl.ds(sb + g * 16, 16)]
        for u in range(16):
          row = lvec[u]
          wsc = wvec[u]
          e = g * 16 + u

          @plsc.parallel_loop(0, D, 16, unroll=D // 16)
          def _(c0):
            msg = rows[slot, e, pl.ds(c0, 16)] * wsc
            cur = acc[row, pl.ds(c0, 16)]
            if is_max:
              acc[row, pl.ds(c0, 16)] = jnp.maximum(cur, msg)
            else:
              acc[row, pl.ds(c0, 16)] = cur + msg
        return 0

      lax.fori_loop(0, KCHUNK // 16, edge_group, 0)

      @pl.when(j + 2 < nchunks)
      def _():
        for c in idx_copies(j + 2, slot):
          c.start()

      return 0

    lax.fori_loop(0, nchunks, chunk, 0)
    pltpu.sync_copy(acc, out_hbm.at[pl.ds(pl.multiple_of(t * RANGE, 8), RANGE)])

  return seg_reduce


_seg_sum = _make_seg_reduce(is_max=False)
_seg_max_agg = _make_seg_reduce(is_max=True)


MM_BLOCK = 1000


def _mlp_body(x_ref, agg_ref, w_ref, b_ref, o_ref, *, relu):
  s = x_ref[...] + agg_ref[...]
  o = jnp.dot(s, w_ref[...], preferred_element_type=jnp.float32) + b_ref[...]
  if relu:
    o = jnp.maximum(o, 0.0)
  o_ref[...] = o


def _mlp(x, agg, w, b, relu):
  return pl.pallas_call(
      functools.partial(_mlp_body, relu=relu),
      grid=(N_NODES // MM_BLOCK,),
      in_specs=[
          pl.BlockSpec((MM_BLOCK, D), lambda i: (i, 0)),
          pl.BlockSpec((MM_BLOCK, D), lambda i: (i, 0)),
          pl.BlockSpec((D, D), lambda i: (0, 0)),
          pl.BlockSpec((1, D), lambda i: (0, 0)),
      ],
      out_specs=pl.BlockSpec((MM_BLOCK, D), lambda i: (i, 0)),
      out_shape=jax.ShapeDtypeStruct((N_NODES, D), jnp.float32),
  )(x, agg, w, b.reshape(1, D))


def kernel(x, edge_index, edge_weight, W1, b1, W2, b2, W3, b3):
  src = edge_index[0]
  dst = edge_index[1]
  bsrc, bldst, bw, cnts = _partition(dst, src, edge_weight)

  agg1 = _seg_sum(x, bsrc, bldst, bw, cnts)[:N_NODES]
  h = _mlp(x, agg1, W1, b1, relu=True)
  agg2 = _seg_max_agg(h, bsrc, bldst, bw, cnts)[:N_NODES]
  h = _mlp(h, agg2, W2, b2, relu=True)
  agg3 = _seg_max_agg(h, bsrc, bldst, bw, cnts)[:N_NODES]
  return _mlp(h, agg3, W3, b3, relu=False)


# speedup vs baseline: 4.4600x; 1.0500x over previous
import functools

import jax
import jax.numpy as jnp
from jax import lax
from jax.experimental import pallas as pl
from jax.experimental.pallas import tpu as pltpu
from jax.experimental.pallas import tpu_sc as plsc

N_NODES = 10000
N_EDGES = 160000
D = 256

NC = 2
NS = 16
NW = NC * NS
RANGE = 320
N_PAD = NW * RANGE

ESLOT = 2048
ISLOT = 128
FLUSH = 2048
PUNROLL = 5
PBUF = 4128
FFLUSH = FLUSH + 80
ROW_W = 159744 + FFLUSH
ECHUNK = 2000
N_ECHUNK = N_EDGES // ECHUNK
KCHUNK = 64

_mesh = plsc.VectorSubcoreMesh(
    core_axis_name="c", subcore_axis_name="s", num_cores=NC, num_subcores=NS)
_sc_params = pltpu.CompilerParams(needs_layout_passes=False)


def _worker_id():
  return lax.axis_index("s") * NC + lax.axis_index("c")


@functools.partial(
    pl.kernel,
    out_type=(
        jax.ShapeDtypeStruct((NW * ROW_W,), jnp.int32),
        jax.ShapeDtypeStruct((NW * ROW_W,), jnp.int32),
        jax.ShapeDtypeStruct((NW * ROW_W,), jnp.float32),
        jax.ShapeDtypeStruct((NW * 16,), jnp.int32),
    ),
    mesh=_mesh,
    scratch_types=[
        pltpu.VMEM((2 * ESLOT,), jnp.int32),
        pltpu.VMEM((2 * ESLOT,), jnp.int32),
        pltpu.VMEM((2 * ESLOT,), jnp.float32),
        pltpu.VMEM((PBUF,), jnp.int32),
        pltpu.VMEM((PBUF,), jnp.int32),
        pltpu.VMEM((PBUF,), jnp.float32),
        pltpu.VMEM((16,), jnp.int32),
        pltpu.SemaphoreType.DMA((2,)),
    ],
    compiler_params=_sc_params,
)
def _partition(dst_hbm, src_hbm, w_hbm, bsrc, bldst, bw, cnts,
               dbuf, sbuf, wbuf, pb_s, pb_l, pb_w, cvec, semc):
  t = _worker_id()
  lo = t * RANGE

  def in_copies(j, slot):
    eoff = pl.multiple_of(j * ECHUNK, 8)
    sbase = pl.multiple_of(slot * ESLOT, 8)
    return (
        pltpu.make_async_copy(dst_hbm.at[pl.ds(eoff, ECHUNK)],
                              dbuf.at[pl.ds(sbase, ECHUNK)], semc.at[slot]),
        pltpu.make_async_copy(src_hbm.at[pl.ds(eoff, ECHUNK)],
                              sbuf.at[pl.ds(sbase, ECHUNK)], semc.at[slot]),
        pltpu.make_async_copy(w_hbm.at[pl.ds(eoff, ECHUNK)],
                              wbuf.at[pl.ds(sbase, ECHUNK)], semc.at[slot]),
    )

  for c in in_copies(0, 0):
    c.start()

  def chunk_body(j, carry):
    slot = j & 1
    nxt = 1 - slot

    @pl.when(j + 1 < N_ECHUNK)
    def _():
      for c in in_copies(j + 1, nxt):
        c.start()

    for c in in_copies(j, slot):
      c.wait()

    sbase = slot * ESLOT
    last16 = jnp.full((16,), 15, jnp.int32)
    cnt_vec, total = carry

    def step(q, cnt_vec):
      for u in range(PUNROLL):
        i = q * PUNROLL + u
        d = dbuf[pl.ds(sbase + i * 16, 16)]
        sv = sbuf[pl.ds(sbase + i * 16, 16)]
        wv = wbuf[pl.ds(sbase + i * 16, 16)]
        m = (d >= lo) & (d < lo + RANGE)
        mi = m.astype(jnp.int32)
        pos = cnt_vec + plsc.cumsum(mi) - 1
        plsc.store_scatter(pb_l, [pos], d - lo, mask=m)
        plsc.store_scatter(pb_s, [pos], sv, mask=m)
        plsc.store_scatter(pb_w, [pos], wv, mask=m)
        cnt_vec = jnp.full((16,), pos[15] + 1, jnp.int32)
      return cnt_vec

    cnt_vec = lax.fori_loop(0, ECHUNK // 16 // PUNROLL, step, cnt_vec)

    cnt_s = jnp.max(cnt_vec)

    def do_flush(args):
      cnt_s, cnt_vec, total = args
      off = pl.multiple_of(t * ROW_W + total, 8)
      pltpu.sync_copy(pb_s.at[pl.ds(0, FLUSH)], bsrc.at[pl.ds(off, FLUSH)])
      pltpu.sync_copy(pb_l.at[pl.ds(0, FLUSH)], bldst.at[pl.ds(off, FLUSH)])
      pltpu.sync_copy(pb_w.at[pl.ds(0, FLUSH)], bw.at[pl.ds(off, FLUSH)])
      rem = cnt_s - FLUSH

      def mv(i, _):
        rs = pb_s[pl.ds(FLUSH + i * 16, 16)]
        rl = pb_l[pl.ds(FLUSH + i * 16, 16)]
        rw = pb_w[pl.ds(FLUSH + i * 16, 16)]
        pb_s[pl.ds(i * 16, 16)] = rs
        pb_l[pl.ds(i * 16, 16)] = rl
        pb_w[pl.ds(i * 16, 16)] = rw
        return 0

      lax.fori_loop(0, (rem + 15) >> 4, mv, 0)
      return (cnt_vec - FLUSH, total + FLUSH)

    return lax.cond(cnt_s >= FLUSH, do_flush,
                    lambda a: (a[1], a[2]), (cnt_s, cnt_vec, total))

  cnt_vec, total = lax.fori_loop(
      0, N_ECHUNK, chunk_body, (jnp.zeros((16,), jnp.int32), jnp.int32(0)))
  cnt = jnp.max(cnt_vec)

  iota = lax.iota(jnp.int32, 16)
  zi = jnp.zeros((16,), jnp.int32)
  zf = jnp.zeros((16,), jnp.float32)
  for k in range(5):
    posz = cnt + k * 16 + iota
    plsc.store_scatter(pb_l, [posz], zi)
    plsc.store_scatter(pb_s, [posz], zi)
    plsc.store_scatter(pb_w, [posz], zf)
  off = pl.multiple_of(t * ROW_W + total, 8)
  pltpu.sync_copy(pb_s.at[pl.ds(0, FFLUSH)], bsrc.at[pl.ds(off, FFLUSH)])
  pltpu.sync_copy(pb_l.at[pl.ds(0, FFLUSH)], bldst.at[pl.ds(off, FFLUSH)])
  pltpu.sync_copy(pb_w.at[pl.ds(0, FFLUSH)], bw.at[pl.ds(off, FFLUSH)])

  cvec[...] = jnp.full((16,), total + cnt, jnp.int32)
  pltpu.sync_copy(cvec, cnts.at[pl.ds(pl.multiple_of(t * 16, 8), 16)])


EUNROLL = 4


def _make_seg_reduce(is_max):
  @functools.partial(
      pl.kernel,
      out_type=jax.ShapeDtypeStruct((N_PAD, D), jnp.float32),
      mesh=_mesh,
      scratch_types=[
          pltpu.VMEM((RANGE, D), jnp.float32),
          pltpu.VMEM((2, KCHUNK, D), jnp.float32),
          pltpu.VMEM((2 * ISLOT,), jnp.int32),
          pltpu.VMEM((2 * ISLOT,), jnp.int32),
          pltpu.VMEM((2 * ISLOT,), jnp.float32),
          pltpu.VMEM((16,), jnp.int32),
          pltpu.SemaphoreType.DMA((2,)),
          pltpu.SemaphoreType.DMA((2,)),
      ],
      compiler_params=_sc_params,
  )
  def seg_reduce(h_hbm, bsrc, bldst, bw, cnts, out_hbm,
                 acc, rows, ib_s, ib_l, ib_w, cbuf, semi, semr):
    t = _worker_id()
    pltpu.sync_copy(cnts.at[pl.ds(pl.multiple_of(t * 16, 8), 16)], cbuf)
    n = jnp.max(cbuf[...])
    nchunks = (n + (KCHUNK - 1)) >> 6

    def idx_copies(j, slot):
      base = pl.multiple_of(t * ROW_W + j * KCHUNK, 8)
      sb = pl.multiple_of(slot * ISLOT, 8)
      return (
          pltpu.make_async_copy(bsrc.at[pl.ds(base, KCHUNK)],
                                ib_s.at[pl.ds(sb, KCHUNK)], semi.at[slot]),
          pltpu.make_async_copy(bldst.at[pl.ds(base, KCHUNK)],
                                ib_l.at[pl.ds(sb, KCHUNK)], semi.at[slot]),
          pltpu.make_async_copy(bw.at[pl.ds(base, KCHUNK)],
                                ib_w.at[pl.ds(sb, KCHUNK)], semi.at[slot]),
      )

    def row_copy(slot):
      sb = pl.multiple_of(slot * ISLOT, 8)
      return pltpu.make_async_copy(h_hbm.at[ib_s.at[pl.ds(sb, KCHUNK)]],
                                   rows.at[slot], semr.at[slot])

    zf = jnp.zeros((16,), jnp.float32)

    @pl.when(nchunks > 0)
    def _():
      for c in idx_copies(0, 0):
        c.start()

    def zrow(i, _):
      for dch in range(D // 16):
        acc[i, pl.ds(dch * 16, 16)] = zf
      return 0

    lax.fori_loop(0, RANGE, zrow, 0)

    @pl.when(nchunks > 0)
    def _():
      for c in idx_copies(0, 0):
        c.wait()
      row_copy(0).start()

    @pl.when(nchunks > 1)
    def _():
      for c in idx_copies(1, 1):
        c.start()

    iota = lax.iota(jnp.int32, 16)

    def chunk(j, _):
      slot = j & 1
      nxt = 1 - slot

      @pl.when(j + 1 < nchunks)
      def _():
        for c in idx_copies(j + 1, nxt):
          c.wait()
        row_copy(nxt).start()

      row_copy(slot).wait()

      sb = slot * ISLOT

      def edge_group(g, _):
        lvec = ib_l[pl.ds(sb + g * 16, 16)]
        wvec = ib_w[pl.ds(sb + g * 16, 16)]
        for u in range(16):
          row = lvec[u]
          wsc = wvec[u]
          e = g * 16 + u

          if is_max:
            @plsc.parallel_loop(0, D, 16, unroll=D // 16)
            def _(c0):
              msg = rows[slot, e, pl.ds(c0, 16)] * wsc
              cur = acc[row, pl.ds(c0, 16)]
              acc[row, pl.ds(c0, 16)] = jnp.maximum(cur, msg)
          else:
            rowsplat = jnp.full((16,), row, jnp.int32)

            @plsc.parallel_loop(0, D, 16, unroll=D // 16)
            def _(c0):
              msg = rows[slot, e, pl.ds(c0, 16)] * wsc
              plsc.addupdate_scatter(acc, [rowsplat, c0 + iota], msg)
        return 0

      lax.fori_loop(0, KCHUNK // 16, edge_group, 0)

      @pl.when(j + 2 < nchunks)
      def _():
        for c in idx_copies(j + 2, slot):
          c.start()

      return 0

    lax.fori_loop(0, nchunks, chunk, 0)
    pltpu.sync_copy(acc, out_hbm.at[pl.ds(pl.multiple_of(t * RANGE, 8), RANGE)])

  return seg_reduce


_seg_sum = _make_seg_reduce(is_max=False)
_seg_max_agg = _make_seg_reduce(is_max=True)


MM_BLOCK = 1000


def _mlp_body(x_ref, agg_ref, w_ref, b_ref, o_ref, *, relu):
  s = x_ref[...] + agg_ref[...]
  o = jnp.dot(s, w_ref[...], preferred_element_type=jnp.float32) + b_ref[...]
  if relu:
    o = jnp.maximum(o, 0.0)
  o_ref[...] = o


def _mlp(x, agg, w, b, relu):
  return pl.pallas_call(
      functools.partial(_mlp_body, relu=relu),
      grid=(N_NODES // MM_BLOCK,),
      in_specs=[
          pl.BlockSpec((MM_BLOCK, D), lambda i: (i, 0)),
          pl.BlockSpec((MM_BLOCK, D), lambda i: (i, 0)),
          pl.BlockSpec((D, D), lambda i: (0, 0)),
          pl.BlockSpec((1, D), lambda i: (0, 0)),
      ],
      out_specs=pl.BlockSpec((MM_BLOCK, D), lambda i: (i, 0)),
      out_shape=jax.ShapeDtypeStruct((N_NODES, D), jnp.float32),
  )(x, agg, w, b.reshape(1, D))


def kernel(x, edge_index, edge_weight, W1, b1, W2, b2, W3, b3):
  src = edge_index[0]
  dst = edge_index[1]
  bsrc, bldst, bw, cnts = _partition(dst, src, edge_weight)

  agg1 = _seg_sum(x, bsrc, bldst, bw, cnts)
  h = _mlp(x, agg1, W1, b1, relu=True)
  agg2 = _seg_max_agg(h, bsrc, bldst, bw, cnts)
  h = _mlp(h, agg2, W2, b2, relu=True)
  agg3 = _seg_max_agg(h, bsrc, bldst, bw, cnts)
  return _mlp(h, agg3, W3, b3, relu=False)


# speedup vs baseline: 4.4688x; 1.0020x over previous
import functools

import jax
import jax.numpy as jnp
from jax import lax
from jax.experimental import pallas as pl
from jax.experimental.pallas import tpu as pltpu
from jax.experimental.pallas import tpu_sc as plsc

N_NODES = 10000
N_EDGES = 160000
D = 256

NC = 2
NS = 16
NW = NC * NS
RANGE = 320
N_PAD = NW * RANGE

ESLOT = 2048
ISLOT = 128
FLUSH = 2048
PUNROLL = 5
PBUF = 4128
FFLUSH = FLUSH + 80
ROW_W = 159744 + FFLUSH
ECHUNK = 2000
N_ECHUNK = N_EDGES // ECHUNK
KCHUNK = 64

_mesh = plsc.VectorSubcoreMesh(
    core_axis_name="c", subcore_axis_name="s", num_cores=NC, num_subcores=NS)
_sc_params = pltpu.CompilerParams(needs_layout_passes=False)


def _worker_id():
  return lax.axis_index("s") * NC + lax.axis_index("c")


@functools.partial(
    pl.kernel,
    out_type=(
        jax.ShapeDtypeStruct((NW * ROW_W,), jnp.int32),
        jax.ShapeDtypeStruct((NW * ROW_W,), jnp.int32),
        jax.ShapeDtypeStruct((NW * ROW_W,), jnp.float32),
        jax.ShapeDtypeStruct((NW * 16,), jnp.int32),
    ),
    mesh=_mesh,
    scratch_types=[
        pltpu.VMEM((2 * ESLOT,), jnp.int32),
        pltpu.VMEM((2 * ESLOT,), jnp.int32),
        pltpu.VMEM((2 * ESLOT,), jnp.float32),
        pltpu.VMEM((PBUF,), jnp.int32),
        pltpu.VMEM((PBUF,), jnp.int32),
        pltpu.VMEM((PBUF,), jnp.float32),
        pltpu.VMEM((16,), jnp.int32),
        pltpu.SemaphoreType.DMA((2,)),
    ],
    compiler_params=_sc_params,
)
def _partition(dst_hbm, src_hbm, w_hbm, bsrc, bldst, bw, cnts,
               dbuf, sbuf, wbuf, pb_s, pb_l, pb_w, cvec, semc):
  t = _worker_id()
  lo = t * RANGE

  def in_copies(j, slot):
    eoff = pl.multiple_of(j * ECHUNK, 8)
    sbase = pl.multiple_of(slot * ESLOT, 8)
    return (
        pltpu.make_async_copy(dst_hbm.at[pl.ds(eoff, ECHUNK)],
                              dbuf.at[pl.ds(sbase, ECHUNK)], semc.at[slot]),
        pltpu.make_async_copy(src_hbm.at[pl.ds(eoff, ECHUNK)],
                              sbuf.at[pl.ds(sbase, ECHUNK)], semc.at[slot]),
        pltpu.make_async_copy(w_hbm.at[pl.ds(eoff, ECHUNK)],
                              wbuf.at[pl.ds(sbase, ECHUNK)], semc.at[slot]),
    )

  for c in in_copies(0, 0):
    c.start()

  def chunk_body(j, carry):
    slot = j & 1
    nxt = 1 - slot

    @pl.when(j + 1 < N_ECHUNK)
    def _():
      for c in in_copies(j + 1, nxt):
        c.start()

    for c in in_copies(j, slot):
      c.wait()

    sbase = slot * ESLOT
    last16 = jnp.full((16,), 15, jnp.int32)
    cnt_vec, total = carry

    def step(q, cnt_vec):
      for u in range(PUNROLL):
        i = q * PUNROLL + u
        d = dbuf[pl.ds(sbase + i * 16, 16)]
        sv = sbuf[pl.ds(sbase + i * 16, 16)]
        wv = wbuf[pl.ds(sbase + i * 16, 16)]
        m = (d >= lo) & (d < lo + RANGE)
        mi = m.astype(jnp.int32)
        pos = cnt_vec + plsc.cumsum(mi) - 1
        plsc.store_scatter(pb_l, [pos], d - lo, mask=m)
        plsc.store_scatter(pb_s, [pos], sv, mask=m)
        plsc.store_scatter(pb_w, [pos], wv, mask=m)
        cnt_vec = jnp.full((16,), pos[15] + 1, jnp.int32)
      return cnt_vec

    cnt_vec = lax.fori_loop(0, ECHUNK // 16 // PUNROLL, step, cnt_vec)

    cnt_s = jnp.max(cnt_vec)

    def do_flush(args):
      cnt_s, cnt_vec, total = args
      off = pl.multiple_of(t * ROW_W + total, 8)
      pltpu.sync_copy(pb_s.at[pl.ds(0, FLUSH)], bsrc.at[pl.ds(off, FLUSH)])
      pltpu.sync_copy(pb_l.at[pl.ds(0, FLUSH)], bldst.at[pl.ds(off, FLUSH)])
      pltpu.sync_copy(pb_w.at[pl.ds(0, FLUSH)], bw.at[pl.ds(off, FLUSH)])
      rem = cnt_s - FLUSH

      def mv(i, _):
        rs = pb_s[pl.ds(FLUSH + i * 16, 16)]
        rl = pb_l[pl.ds(FLUSH + i * 16, 16)]
        rw = pb_w[pl.ds(FLUSH + i * 16, 16)]
        pb_s[pl.ds(i * 16, 16)] = rs
        pb_l[pl.ds(i * 16, 16)] = rl
        pb_w[pl.ds(i * 16, 16)] = rw
        return 0

      lax.fori_loop(0, (rem + 15) >> 4, mv, 0)
      return (cnt_vec - FLUSH, total + FLUSH)

    return lax.cond(cnt_s >= FLUSH, do_flush,
                    lambda a: (a[1], a[2]), (cnt_s, cnt_vec, total))

  cnt_vec, total = lax.fori_loop(
      0, N_ECHUNK, chunk_body, (jnp.zeros((16,), jnp.int32), jnp.int32(0)))
  cnt = jnp.max(cnt_vec)

  iota = lax.iota(jnp.int32, 16)
  zi = jnp.zeros((16,), jnp.int32)
  zf = jnp.zeros((16,), jnp.float32)
  for k in range(5):
    posz = cnt + k * 16 + iota
    plsc.store_scatter(pb_l, [posz], zi)
    plsc.store_scatter(pb_s, [posz], zi)
    plsc.store_scatter(pb_w, [posz], zf)
  off = pl.multiple_of(t * ROW_W + total, 8)
  pltpu.sync_copy(pb_s.at[pl.ds(0, FFLUSH)], bsrc.at[pl.ds(off, FFLUSH)])
  pltpu.sync_copy(pb_l.at[pl.ds(0, FFLUSH)], bldst.at[pl.ds(off, FFLUSH)])
  pltpu.sync_copy(pb_w.at[pl.ds(0, FFLUSH)], bw.at[pl.ds(off, FFLUSH)])

  cvec[...] = jnp.full((16,), total + cnt, jnp.int32)
  pltpu.sync_copy(cvec, cnts.at[pl.ds(pl.multiple_of(t * 16, 8), 16)])


EUNROLL = 4

_LANE_DNUMS = lax.GatherDimensionNumbers(
    offset_dims=(), collapsed_slice_dims=(0,), start_index_map=(0,))


def _make_seg_reduce(is_max):
  @functools.partial(
      pl.kernel,
      out_type=jax.ShapeDtypeStruct((N_PAD, D), jnp.float32),
      mesh=_mesh,
      scratch_types=[
          pltpu.VMEM((RANGE, D), jnp.float32),
          pltpu.VMEM((2, KCHUNK, D), jnp.float32),
          pltpu.VMEM((2 * ISLOT,), jnp.int32),
          pltpu.VMEM((2 * ISLOT,), jnp.int32),
          pltpu.VMEM((2 * ISLOT,), jnp.float32),
          pltpu.VMEM((16,), jnp.int32),
          pltpu.SemaphoreType.DMA((2,)),
          pltpu.SemaphoreType.DMA((2,)),
      ],
      compiler_params=_sc_params,
  )
  def seg_reduce(h_hbm, bsrc, bldst, bw, cnts, out_hbm,
                 acc, rows, ib_s, ib_l, ib_w, cbuf, semi, semr):
    t = _worker_id()
    pltpu.sync_copy(cnts.at[pl.ds(pl.multiple_of(t * 16, 8), 16)], cbuf)
    n = jnp.max(cbuf[...])
    nchunks = (n + (KCHUNK - 1)) >> 6

    def idx_copies(j, slot):
      base = pl.multiple_of(t * ROW_W + j * KCHUNK, 8)
      sb = pl.multiple_of(slot * ISLOT, 8)
      return (
          pltpu.make_async_copy(bsrc.at[pl.ds(base, KCHUNK)],
                                ib_s.at[pl.ds(sb, KCHUNK)], semi.at[slot]),
          pltpu.make_async_copy(bldst.at[pl.ds(base, KCHUNK)],
                                ib_l.at[pl.ds(sb, KCHUNK)], semi.at[slot]),
          pltpu.make_async_copy(bw.at[pl.ds(base, KCHUNK)],
                                ib_w.at[pl.ds(sb, KCHUNK)], semi.at[slot]),
      )

    def row_copy(slot):
      sb = pl.multiple_of(slot * ISLOT, 8)
      return pltpu.make_async_copy(h_hbm.at[ib_s.at[pl.ds(sb, KCHUNK)]],
                                   rows.at[slot], semr.at[slot])

    zf = jnp.zeros((16,), jnp.float32)

    @pl.when(nchunks > 0)
    def _():
      for c in idx_copies(0, 0):
        c.start()

    def zrow(i, _):
      for dch in range(D // 16):
        acc[i, pl.ds(dch * 16, 16)] = zf
      return 0

    lax.fori_loop(0, RANGE, zrow, 0)

    @pl.when(nchunks > 0)
    def _():
      for c in idx_copies(0, 0):
        c.wait()
      row_copy(0).start()

    @pl.when(nchunks > 1)
    def _():
      for c in idx_copies(1, 1):
        c.start()

    iota = lax.iota(jnp.int32, 16)

    def chunk(j, _):
      slot = j & 1
      nxt = 1 - slot

      @pl.when(j + 1 < nchunks)
      def _():
        for c in idx_copies(j + 1, nxt):
          c.wait()
        row_copy(nxt).start()

      row_copy(slot).wait()

      sb = slot * ISLOT

      def edge_group(g, _):
        lvec = ib_l[pl.ds(sb + g * 16, 16)]
        wvec = ib_w[pl.ds(sb + g * 16, 16)]
        if is_max:
          for u in range(16):
            row = lvec[u]
            wsc = wvec[u]
            e = g * 16 + u

            @plsc.parallel_loop(0, D, 16, unroll=D // 16)
            def _(c0):
              msg = rows[slot, e, pl.ds(c0, 16)] * wsc
              cur = acc[row, pl.ds(c0, 16)]
              acc[row, pl.ds(c0, 16)] = jnp.maximum(cur, msg)
        else:
          e0 = g * 16

          @plsc.parallel_loop(0, 256, 1, unroll=16)
          def _(i):
            u = i >> 4
            c0 = (i & 15) * 16
            usplat = jnp.full((16, 1), u, jnp.int32)
            lsp = lax.gather(
                lvec, usplat, _LANE_DNUMS, (1,),
                mode=lax.GatherScatterMode.PROMISE_IN_BOUNDS)
            wsp = lax.gather(
                wvec, usplat, _LANE_DNUMS, (1,),
                mode=lax.GatherScatterMode.PROMISE_IN_BOUNDS)
            msg = rows[slot, e0 + u, pl.ds(c0, 16)] * wsp
            plsc.addupdate_scatter(acc, [lsp, c0 + iota], msg)
        return 0

      lax.fori_loop(0, KCHUNK // 16, edge_group, 0)

      @pl.when(j + 2 < nchunks)
      def _():
        for c in idx_copies(j + 2, slot):
          c.start()

      return 0

    lax.fori_loop(0, nchunks, chunk, 0)
    pltpu.sync_copy(acc, out_hbm.at[pl.ds(pl.multiple_of(t * RANGE, 8), RANGE)])

  return seg_reduce


_seg_sum = _make_seg_reduce(is_max=False)
_seg_max_agg = _make_seg_reduce(is_max=True)


MM_BLOCK = 1000


def _mlp_body(x_ref, agg_ref, w_ref, b_ref, o_ref, *, relu):
  s = x_ref[...] + agg_ref[...]
  o = jnp.dot(s, w_ref[...], preferred_element_type=jnp.float32) + b_ref[...]
  if relu:
    o = jnp.maximum(o, 0.0)
  o_ref[...] = o


def _mlp(x, agg, w, b, relu):
  return pl.pallas_call(
      functools.partial(_mlp_body, relu=relu),
      grid=(N_NODES // MM_BLOCK,),
      in_specs=[
          pl.BlockSpec((MM_BLOCK, D), lambda i: (i, 0)),
          pl.BlockSpec((MM_BLOCK, D), lambda i: (i, 0)),
          pl.BlockSpec((D, D), lambda i: (0, 0)),
          pl.BlockSpec((1, D), lambda i: (0, 0)),
      ],
      out_specs=pl.BlockSpec((MM_BLOCK, D), lambda i: (i, 0)),
      out_shape=jax.ShapeDtypeStruct((N_NODES, D), jnp.float32),
  )(x, agg, w, b.reshape(1, D))


def kernel(x, edge_index, edge_weight, W1, b1, W2, b2, W3, b3):
  src = edge_index[0]
  dst = edge_index[1]
  bsrc, bldst, bw, cnts = _partition(dst, src, edge_weight)

  agg1 = _seg_sum(x, bsrc, bldst, bw, cnts)
  h = _mlp(x, agg1, W1, b1, relu=True)
  agg2 = _seg_max_agg(h, bsrc, bldst, bw, cnts)
  h = _mlp(h, agg2, W2, b2, relu=True)
  agg3 = _seg_max_agg(h, bsrc, bldst, bw, cnts)
  return _mlp(h, agg3, W3, b3, relu=False)


# speedup vs baseline: 4.4692x; 1.0001x over previous
import functools

import jax
import jax.numpy as jnp
from jax import lax
from jax.experimental import pallas as pl
from jax.experimental.pallas import tpu as pltpu
from jax.experimental.pallas import tpu_sc as plsc

N_NODES = 10000
N_EDGES = 160000
D = 256

NC = 2
NS = 16
NW = NC * NS
RANGE = 320
N_PAD = NW * RANGE

ESLOT = 2048
ISLOT = 128
FLUSH = 2048
PUNROLL = 5
PBUF = 4128
FFLUSH = FLUSH + 80
ROW_W = 159744 + FFLUSH
ECHUNK = 2000
N_ECHUNK = N_EDGES // ECHUNK
KCHUNK = 64

_mesh = plsc.VectorSubcoreMesh(
    core_axis_name="c", subcore_axis_name="s", num_cores=NC, num_subcores=NS)
_sc_params = pltpu.CompilerParams(needs_layout_passes=False)


def _worker_id():
  return lax.axis_index("s") * NC + lax.axis_index("c")


@functools.partial(
    pl.kernel,
    out_type=(
        jax.ShapeDtypeStruct((NW * ROW_W,), jnp.int32),
        jax.ShapeDtypeStruct((NW * ROW_W,), jnp.int32),
        jax.ShapeDtypeStruct((NW * ROW_W,), jnp.float32),
        jax.ShapeDtypeStruct((NW * 16,), jnp.int32),
    ),
    mesh=_mesh,
    scratch_types=[
        pltpu.VMEM((2 * ESLOT,), jnp.int32),
        pltpu.VMEM((2 * ESLOT,), jnp.int32),
        pltpu.VMEM((2 * ESLOT,), jnp.float32),
        pltpu.VMEM((PBUF,), jnp.int32),
        pltpu.VMEM((PBUF,), jnp.int32),
        pltpu.VMEM((PBUF,), jnp.float32),
        pltpu.VMEM((16,), jnp.int32),
        pltpu.SemaphoreType.DMA((2,)),
    ],
    compiler_params=_sc_params,
)
def _partition(dst_hbm, src_hbm, w_hbm, bsrc, bldst, bw, cnts,
               dbuf, sbuf, wbuf, pb_s, pb_l, pb_w, cvec, semc):
  t = _worker_id()
  lo = t * RANGE

  def in_copies(j, slot):
    eoff = pl.multiple_of(j * ECHUNK, 8)
    sbase = pl.multiple_of(slot * ESLOT, 8)
    return (
        pltpu.make_async_copy(dst_hbm.at[pl.ds(eoff, ECHUNK)],
                              dbuf.at[pl.ds(sbase, ECHUNK)], semc.at[slot]),
        pltpu.make_async_copy(src_hbm.at[pl.ds(eoff, ECHUNK)],
                              sbuf.at[pl.ds(sbase, ECHUNK)], semc.at[slot]),
        pltpu.make_async_copy(w_hbm.at[pl.ds(eoff, ECHUNK)],
                              wbuf.at[pl.ds(sbase, ECHUNK)], semc.at[slot]),
    )

  for c in in_copies(0, 0):
    c.start()

  def chunk_body(j, carry):
    slot = j & 1
    nxt = 1 - slot

    @pl.when(j + 1 < N_ECHUNK)
    def _():
      for c in in_copies(j + 1, nxt):
        c.start()

    for c in in_copies(j, slot):
      c.wait()

    sbase = slot * ESLOT
    last16 = jnp.full((16,), 15, jnp.int32)
    cnt_vec, total = carry

    def step(q, cnt_vec):
      for u in range(PUNROLL):
        i = q * PUNROLL + u
        d = dbuf[pl.ds(sbase + i * 16, 16)]
        sv = sbuf[pl.ds(sbase + i * 16, 16)]
        wv = wbuf[pl.ds(sbase + i * 16, 16)]
        m = (d >= lo) & (d < lo + RANGE)
        mi = m.astype(jnp.int32)
        pos = cnt_vec + plsc.cumsum(mi) - 1
        plsc.store_scatter(pb_l, [pos], d - lo, mask=m)
        plsc.store_scatter(pb_s, [pos], sv, mask=m)
        plsc.store_scatter(pb_w, [pos], wv, mask=m)
        cnt_vec = jnp.full((16,), pos[15] + 1, jnp.int32)
      return cnt_vec

    cnt_vec = lax.fori_loop(0, ECHUNK // 16 // PUNROLL, step, cnt_vec)

    cnt_s = jnp.max(cnt_vec)

    def do_flush(args):
      cnt_s, cnt_vec, total = args
      off = pl.multiple_of(t * ROW_W + total, 8)
      pltpu.sync_copy(pb_s.at[pl.ds(0, FLUSH)], bsrc.at[pl.ds(off, FLUSH)])
      pltpu.sync_copy(pb_l.at[pl.ds(0, FLUSH)], bldst.at[pl.ds(off, FLUSH)])
      pltpu.sync_copy(pb_w.at[pl.ds(0, FLUSH)], bw.at[pl.ds(off, FLUSH)])
      rem = cnt_s - FLUSH

      def mv(i, _):
        rs = pb_s[pl.ds(FLUSH + i * 16, 16)]
        rl = pb_l[pl.ds(FLUSH + i * 16, 16)]
        rw = pb_w[pl.ds(FLUSH + i * 16, 16)]
        pb_s[pl.ds(i * 16, 16)] = rs
        pb_l[pl.ds(i * 16, 16)] = rl
        pb_w[pl.ds(i * 16, 16)] = rw
        return 0

      lax.fori_loop(0, (rem + 15) >> 4, mv, 0)
      return (cnt_vec - FLUSH, total + FLUSH)

    return lax.cond(cnt_s >= FLUSH, do_flush,
                    lambda a: (a[1], a[2]), (cnt_s, cnt_vec, total))

  cnt_vec, total = lax.fori_loop(
      0, N_ECHUNK, chunk_body, (jnp.zeros((16,), jnp.int32), jnp.int32(0)))
  cnt = jnp.max(cnt_vec)

  iota = lax.iota(jnp.int32, 16)
  zi = jnp.zeros((16,), jnp.int32)
  zf = jnp.zeros((16,), jnp.float32)
  for k in range(5):
    posz = cnt + k * 16 + iota
    plsc.store_scatter(pb_l, [posz], zi)
    plsc.store_scatter(pb_s, [posz], zi)
    plsc.store_scatter(pb_w, [posz], zf)
  off = pl.multiple_of(t * ROW_W + total, 8)
  pltpu.sync_copy(pb_s.at[pl.ds(0, FFLUSH)], bsrc.at[pl.ds(off, FFLUSH)])
  pltpu.sync_copy(pb_l.at[pl.ds(0, FFLUSH)], bldst.at[pl.ds(off, FFLUSH)])
  pltpu.sync_copy(pb_w.at[pl.ds(0, FFLUSH)], bw.at[pl.ds(off, FFLUSH)])

  cvec[...] = jnp.full((16,), total + cnt, jnp.int32)
  pltpu.sync_copy(cvec, cnts.at[pl.ds(pl.multiple_of(t * 16, 8), 16)])


EUNROLL = 4

_LANE_DNUMS = lax.GatherDimensionNumbers(
    offset_dims=(), collapsed_slice_dims=(0,), start_index_map=(0,))


def _make_seg_reduce(is_max):
  @functools.partial(
      pl.kernel,
      out_type=jax.ShapeDtypeStruct((N_PAD, D), jnp.float32),
      mesh=_mesh,
      scratch_types=[
          pltpu.VMEM((RANGE, D), jnp.float32),
          pltpu.VMEM((2, KCHUNK, D), jnp.float32),
          pltpu.VMEM((2 * ISLOT,), jnp.int32),
          pltpu.VMEM((2 * ISLOT,), jnp.int32),
          pltpu.VMEM((2 * ISLOT,), jnp.float32),
          pltpu.VMEM((16,), jnp.int32),
          pltpu.SemaphoreType.DMA((2,)),
          pltpu.SemaphoreType.DMA((2,)),
      ],
      compiler_params=_sc_params,
  )
  def seg_reduce(h_hbm, bsrc, bldst, bw, cnts, out_hbm,
                 acc, rows, ib_s, ib_l, ib_w, cbuf, semi, semr):
    t = _worker_id()
    pltpu.sync_copy(cnts.at[pl.ds(pl.multiple_of(t * 16, 8), 16)], cbuf)
    n = jnp.max(cbuf[...])
    nchunks = (n + (KCHUNK - 1)) >> 6

    def idx_copies(j, slot):
      base = pl.multiple_of(t * ROW_W + j * KCHUNK, 8)
      sb = pl.multiple_of(slot * ISLOT, 8)
      return (
          pltpu.make_async_copy(bsrc.at[pl.ds(base, KCHUNK)],
                                ib_s.at[pl.ds(sb, KCHUNK)], semi.at[slot]),
          pltpu.make_async_copy(bldst.at[pl.ds(base, KCHUNK)],
                                ib_l.at[pl.ds(sb, KCHUNK)], semi.at[slot]),
          pltpu.make_async_copy(bw.at[pl.ds(base, KCHUNK)],
                                ib_w.at[pl.ds(sb, KCHUNK)], semi.at[slot]),
      )

    def row_copy(slot):
      sb = pl.multiple_of(slot * ISLOT, 8)
      return pltpu.make_async_copy(h_hbm.at[ib_s.at[pl.ds(sb, KCHUNK)]],
                                   rows.at[slot], semr.at[slot])

    zf = jnp.zeros((16,), jnp.float32)

    @pl.when(nchunks > 0)
    def _():
      for c in idx_copies(0, 0):
        c.start()

    def zrow(i, _):
      for dch in range(D // 16):
        acc[i, pl.ds(dch * 16, 16)] = zf
      return 0

    lax.fori_loop(0, RANGE, zrow, 0)

    @pl.when(nchunks > 0)
    def _():
      for c in idx_copies(0, 0):
        c.wait()
      row_copy(0).start()

    @pl.when(nchunks > 1)
    def _():
      for c in idx_copies(1, 1):
        c.start()

    iota = lax.iota(jnp.int32, 16)

    def chunk(j, _):
      slot = j & 1
      nxt = 1 - slot

      @pl.when(j + 1 < nchunks)
      def _():
        for c in idx_copies(j + 1, nxt):
          c.wait()
        row_copy(nxt).start()

      row_copy(slot).wait()

      sb = slot * ISLOT

      def edge_group(g, _):
        lvec = ib_l[pl.ds(sb + g * 16, 16)]
        wvec = ib_w[pl.ds(sb + g * 16, 16)]
        if is_max:
          for u in range(16):
            usplat = jnp.full((16, 1), u, jnp.int32)
            lsp = lax.gather(
                lvec, usplat, _LANE_DNUMS, (1,),
                mode=lax.GatherScatterMode.PROMISE_IN_BOUNDS)
            wsp = lax.gather(
                wvec, usplat, _LANE_DNUMS, (1,),
                mode=lax.GatherScatterMode.PROMISE_IN_BOUNDS)
            e = g * 16 + u

            @plsc.parallel_loop(0, D, 16, unroll=D // 16)
            def _(c0):
              col = c0 + iota
              msg = rows[slot, e, pl.ds(c0, 16)] * wsp
              cur = plsc.load_gather(acc, [lsp, col])
              plsc.store_scatter(acc, [lsp, col], jnp.maximum(cur, msg))
        else:
          e0 = g * 16

          @plsc.parallel_loop(0, 256, 1, unroll=16)
          def _(i):
            u = i >> 4
            c0 = (i & 15) * 16
            usplat = jnp.full((16, 1), u, jnp.int32)
            lsp = lax.gather(
                lvec, usplat, _LANE_DNUMS, (1,),
                mode=lax.GatherScatterMode.PROMISE_IN_BOUNDS)
            wsp = lax.gather(
                wvec, usplat, _LANE_DNUMS, (1,),
                mode=lax.GatherScatterMode.PROMISE_IN_BOUNDS)
            msg = rows[slot, e0 + u, pl.ds(c0, 16)] * wsp
            plsc.addupdate_scatter(acc, [lsp, c0 + iota], msg)
        return 0

      lax.fori_loop(0, KCHUNK // 16, edge_group, 0)

      @pl.when(j + 2 < nchunks)
      def _():
        for c in idx_copies(j + 2, slot):
          c.start()

      return 0

    lax.fori_loop(0, nchunks, chunk, 0)
    pltpu.sync_copy(acc, out_hbm.at[pl.ds(pl.multiple_of(t * RANGE, 8), RANGE)])

  return seg_reduce


_seg_sum = _make_seg_reduce(is_max=False)
_seg_max_agg = _make_seg_reduce(is_max=True)


MM_BLOCK = 1000


def _mlp_body(x_ref, agg_ref, w_ref, b_ref, o_ref, *, relu):
  s = x_ref[...] + agg_ref[...]
  o = jnp.dot(s, w_ref[...], preferred_element_type=jnp.float32) + b_ref[...]
  if relu:
    o = jnp.maximum(o, 0.0)
  o_ref[...] = o


def _mlp(x, agg, w, b, relu):
  return pl.pallas_call(
      functools.partial(_mlp_body, relu=relu),
      grid=(N_NODES // MM_BLOCK,),
      in_specs=[
          pl.BlockSpec((MM_BLOCK, D), lambda i: (i, 0)),
          pl.BlockSpec((MM_BLOCK, D), lambda i: (i, 0)),
          pl.BlockSpec((D, D), lambda i: (0, 0)),
          pl.BlockSpec((1, D), lambda i: (0, 0)),
      ],
      out_specs=pl.BlockSpec((MM_BLOCK, D), lambda i: (i, 0)),
      out_shape=jax.ShapeDtypeStruct((N_NODES, D), jnp.float32),
  )(x, agg, w, b.reshape(1, D))


def kernel(x, edge_index, edge_weight, W1, b1, W2, b2, W3, b3):
  src = edge_index[0]
  dst = edge_index[1]
  bsrc, bldst, bw, cnts = _partition(dst, src, edge_weight)

  agg1 = _seg_sum(x, bsrc, bldst, bw, cnts)
  h = _mlp(x, agg1, W1, b1, relu=True)
  agg2 = _seg_max_agg(h, bsrc, bldst, bw, cnts)
  h = _mlp(h, agg2, W2, b2, relu=True)
  agg3 = _seg_max_agg(h, bsrc, bldst, bw, cnts)
  return _mlp(h, agg3, W3, b3, relu=False)
